# baseline (speedup 1.0000x reference)
import jax
import jax.numpy as jnp
from jax.experimental import pallas as pl
from jax.experimental.pallas import tpu as pltpu

K_DOCS = 1_000_000
D = 64
BLK = 4_000
NB = K_DOCS // BLK
NCAND = 32
TOPK = 10
_IMAX = 2**31 - 1


def _score_body(q_ref, d_ref, cand_ref, s_ref):
    i = pl.program_id(0)
    d = d_ref[...]
    q = q_ref[...]
    qn = jnp.sum(q * q)
    dot = jax.lax.dot_general(q, d, (((1,), (1,)), ((), ())),
                              preferred_element_type=jnp.float32)
    nrm = jax.lax.dot_general(q * 0 + 1.0, d * d, (((1,), (1,)), ((), ())),
                              preferred_element_type=jnp.float32)
    s_ref[pl.ds(i, 1), :] = dot / (jnp.sqrt(nrm) * jnp.sqrt(qn))

    @pl.when(i == NB - 1)
    def _():
        rm = jnp.max(s_ref[...], axis=1, keepdims=True)
        riota = jax.lax.broadcasted_iota(jnp.int32, (NB, 1), 0)
        ciota = jax.lax.broadcasted_iota(jnp.int32, (1, BLK), 1)
        lane = jax.lax.broadcasted_iota(jnp.int32, (1, NCAND), 1)
        ivec = jnp.zeros((1, NCAND), jnp.int32)
        for j in range(NCAND):
            m = jnp.max(rm)
            r = jnp.min(jnp.where(rm == m, riota, _IMAX))
            row = s_ref[pl.ds(r, 1), :]
            c = jnp.min(jnp.where(row == m, ciota, _IMAX))
            ivec = jnp.where(lane == j, r * BLK + c, ivec)
            nrow = jnp.where(ciota == c, -jnp.inf, row)
            s_ref[pl.ds(r, 1), :] = nrow
            rm = jnp.where(riota == r, jnp.max(nrow), rm)
        cand_ref[...] = ivec


def _rescore_body(cand_sref, q_ref, d_ref, vals_ref, idx_ref, vs_ref, gs_ref):
    i = pl.program_id(0)
    g = cand_sref[i]
    d8 = d_ref[...]
    q = q_ref[...]
    qn = jnp.sum(jnp.maximum(q * q, 1e-12))
    dot8 = jnp.sum(d8 * q, axis=1, keepdims=True)
    nrm8 = jnp.sum(jnp.maximum(d8 * d8, 1e-12), axis=1, keepdims=True)
    cos8 = dot8 / (jnp.sqrt(nrm8) * jnp.sqrt(qn))
    sub = g - (g // 8) * 8
    sel = jax.lax.broadcasted_iota(jnp.int32, (8, 1), 0) == sub
    v = jnp.max(jnp.where(sel, cos8, -jnp.inf))

    @pl.when(i == 0)
    def _():
        vs_ref[...] = jnp.full((1, NCAND), -jnp.inf, jnp.float32)
        gs_ref[...] = jnp.zeros((1, NCAND), jnp.int32)

    laneC = jax.lax.broadcasted_iota(jnp.int32, (1, NCAND), 1)
    vs_ref[...] = jnp.where(laneC == i, v, vs_ref[...])
    gs_ref[...] = jnp.where(laneC == i, g, gs_ref[...])

    @pl.when(i == NCAND - 1)
    def _():
        vs = vs_ref[...]
        gs = gs_ref[...]
        lane16 = jax.lax.broadcasted_iota(jnp.int32, (1, 16), 1)
        vvec = jnp.full((1, 16), -jnp.inf, jnp.float32)
        ivec = jnp.zeros((1, 16), jnp.int32)
        for j in range(TOPK):
            m = jnp.max(vs)
            gsel = jnp.min(jnp.where(vs == m, gs, _IMAX))
            vvec = jnp.where(lane16 == j, m, vvec)
            ivec = jnp.where(lane16 == j, gsel, ivec)
            vs = jnp.where(gs == gsel, -jnp.inf, vs)
        vals_ref[...] = vvec
        idx_ref[...] = ivec


def kernel(query, docs):
    cand = pl.pallas_call(
        _score_body,
        grid=(NB,),
        in_specs=[
            pl.BlockSpec((1, D), lambda i: (0, 0)),
            pl.BlockSpec((BLK, D), lambda i: (i, 0)),
        ],
        out_specs=pl.BlockSpec((1, NCAND), lambda i: (0, 0)),
        out_shape=jax.ShapeDtypeStruct((1, NCAND), jnp.int32),
        scratch_shapes=[pltpu.VMEM((NB, BLK), jnp.float32)],
    )(query, docs)

    vals, idx = pl.pallas_call(
        _rescore_body,
        grid_spec=pltpu.PrefetchScalarGridSpec(
            num_scalar_prefetch=1,
            grid=(NCAND,),
            in_specs=[
                pl.BlockSpec((1, D), lambda i, cand_s: (0, 0)),
                pl.BlockSpec((8, D), lambda i, cand_s: (cand_s[i] // 8, 0)),
            ],
            out_specs=[
                pl.BlockSpec((1, 16), lambda i, cand_s: (0, 0)),
                pl.BlockSpec((1, 16), lambda i, cand_s: (0, 0)),
            ],
            scratch_shapes=[
                pltpu.VMEM((1, NCAND), jnp.float32),
                pltpu.VMEM((1, NCAND), jnp.int32),
            ],
        ),
        out_shape=[
            jax.ShapeDtypeStruct((1, 16), jnp.float32),
            jax.ShapeDtypeStruct((1, 16), jnp.int32),
        ],
    )(cand.reshape(NCAND), query, docs)
    return vals[0, :TOPK], idx[0, :TOPK]


# speedup vs baseline: 1.0979x; 1.0979x over previous
import jax
import jax.numpy as jnp
from jax.experimental import pallas as pl
from jax.experimental.pallas import tpu as pltpu

K_DOCS = 1_000_000
D = 64
BLK = 8_000
NB = K_DOCS // BLK
NCAND = 32
TOPK = 10
_IMAX = 2**31 - 1


def _score_body(q_ref, d_ref, cand_ref, s_ref, rm_ref):
    i = pl.program_id(0)
    d = d_ref[...]
    q = q_ref[...]
    qn = jnp.sum(q * q)
    dot = jax.lax.dot_general(q, d, (((1,), (1,)), ((), ())),
                              preferred_element_type=jnp.float32)
    nrm = jax.lax.dot_general(q * 0 + 1.0, d * d, (((1,), (1,)), ((), ())),
                              preferred_element_type=jnp.float32)
    cos = dot / (jnp.sqrt(nrm) * jnp.sqrt(qn))
    s_ref[pl.ds(i, 1), :] = cos
    rm_ref[pl.ds(i, 1), :] = jnp.max(cos).reshape(1, 1)

    @pl.when(i == NB - 1)
    def _():
        rm = rm_ref[...]
        riota = jax.lax.broadcasted_iota(jnp.int32, (NB, 1), 0)
        ciota = jax.lax.broadcasted_iota(jnp.int32, (1, BLK), 1)
        lane = jax.lax.broadcasted_iota(jnp.int32, (1, NCAND), 1)
        ivec = jnp.zeros((1, NCAND), jnp.int32)
        for j in range(NCAND):
            m = jnp.max(rm)
            r = jnp.min(jnp.where(rm == m, riota, _IMAX))
            row = s_ref[pl.ds(r, 1), :]
            c = jnp.min(jnp.where(row == m, ciota, _IMAX))
            ivec = jnp.where(lane == j, r * BLK + c, ivec)
            nrow = jnp.where(ciota == c, -jnp.inf, row)
            s_ref[pl.ds(r, 1), :] = nrow
            rm = jnp.where(riota == r, jnp.max(nrow), rm)
        cand_ref[...] = ivec


def _rescore_body(cand_sref, q_ref, d_ref, vals_ref, idx_ref, vs_ref, gs_ref):
    i = pl.program_id(0)
    g = cand_sref[i]
    d8 = d_ref[...]
    q = q_ref[...]
    qn = jnp.sum(jnp.maximum(q * q, 1e-12))
    dot8 = jnp.sum(d8 * q, axis=1, keepdims=True)
    nrm8 = jnp.sum(jnp.maximum(d8 * d8, 1e-12), axis=1, keepdims=True)
    cos8 = dot8 / (jnp.sqrt(nrm8) * jnp.sqrt(qn))
    sub = g - (g // 8) * 8
    sel = jax.lax.broadcasted_iota(jnp.int32, (8, 1), 0) == sub
    v = jnp.max(jnp.where(sel, cos8, -jnp.inf))

    @pl.when(i == 0)
    def _():
        vs_ref[...] = jnp.full((1, NCAND), -jnp.inf, jnp.float32)
        gs_ref[...] = jnp.zeros((1, NCAND), jnp.int32)

    laneC = jax.lax.broadcasted_iota(jnp.int32, (1, NCAND), 1)
    vs_ref[...] = jnp.where(laneC == i, v, vs_ref[...])
    gs_ref[...] = jnp.where(laneC == i, g, gs_ref[...])

    @pl.when(i == NCAND - 1)
    def _():
        vs = vs_ref[...]
        gs = gs_ref[...]
        lane16 = jax.lax.broadcasted_iota(jnp.int32, (1, 16), 1)
        vvec = jnp.full((1, 16), -jnp.inf, jnp.float32)
        ivec = jnp.zeros((1, 16), jnp.int32)
        for j in range(TOPK):
            m = jnp.max(vs)
            gsel = jnp.min(jnp.where(vs == m, gs, _IMAX))
            vvec = jnp.where(lane16 == j, m, vvec)
            ivec = jnp.where(lane16 == j, gsel, ivec)
            vs = jnp.where(gs == gsel, -jnp.inf, vs)
        vals_ref[...] = vvec
        idx_ref[...] = ivec


def kernel(query, docs):
    cand = pl.pallas_call(
        _score_body,
        grid=(NB,),
        in_specs=[
            pl.BlockSpec((1, D), lambda i: (0, 0)),
            pl.BlockSpec((BLK, D), lambda i: (i, 0)),
        ],
        out_specs=pl.BlockSpec((1, NCAND), lambda i: (0, 0)),
        out_shape=jax.ShapeDtypeStruct((1, NCAND), jnp.int32),
        scratch_shapes=[pltpu.VMEM((NB, BLK), jnp.float32),
                        pltpu.VMEM((NB, 1), jnp.float32)],
    )(query, docs)

    vals, idx = pl.pallas_call(
        _rescore_body,
        grid_spec=pltpu.PrefetchScalarGridSpec(
            num_scalar_prefetch=1,
            grid=(NCAND,),
            in_specs=[
                pl.BlockSpec((1, D), lambda i, cand_s: (0, 0)),
                pl.BlockSpec((8, D), lambda i, cand_s: (cand_s[i] // 8, 0)),
            ],
            out_specs=[
                pl.BlockSpec((1, 16), lambda i, cand_s: (0, 0)),
                pl.BlockSpec((1, 16), lambda i, cand_s: (0, 0)),
            ],
            scratch_shapes=[
                pltpu.VMEM((1, NCAND), jnp.float32),
                pltpu.VMEM((1, NCAND), jnp.int32),
            ],
        ),
        out_shape=[
            jax.ShapeDtypeStruct((1, 16), jnp.float32),
            jax.ShapeDtypeStruct((1, 16), jnp.int32),
        ],
    )(cand.reshape(NCAND), query, docs)
    return vals[0, :TOPK], idx[0, :TOPK]
